# baseline (device time: 108279 ns/iter reference)
import jax
import jax.numpy as jnp
from jax import lax
from jax.experimental import pallas as pl
from jax.experimental.pallas import tpu as pltpu

B, S, H, Dh, Dr = 4, 256, 32, 128, 64
D = 4096
DCS = 128
HD = H * Dh
HD2 = HD // 2
SCALE = (Dh + Dr) ** -0.5
BF16 = jnp.bfloat16
F32 = jnp.float32
GPZ = 8
GW = HD2 // GPZ
GRW = GW // Dh * Dr
NJ = 32
SW = D // NJ
SW2 = SW // 2

_MESH = pl.DeviceIdType.MESH


def _main_body(x_hbm, wdkv_ref, wuk_ref, wuv_ref, wkr_ref,
               wq_hbm, wqr_hbm, wo_hbm, out_ref,
               xq32, xb, c_loc, c_rem, wuk_loc, wuk_rem, wuv_loc, wuv_rem,
               kbuf, vbuf, kr_scr, qall, qrall, o_all,
               wq_stage, wqr_stage, wo_stage,
               copy_sem, wq_sems, wqr_sems, wo_sems,
               zsend, zrecv, osend, orecv, gsend, grecv):
    my_x = lax.axis_index("x")
    my_y = lax.axis_index("y")
    my_z = lax.axis_index("z")
    q = 2 * my_x + my_y
    q_xn = 2 * (1 - my_x) + my_y
    q_yn = 2 * my_x + (1 - my_y)
    zp = (my_x, my_y, 1 - my_z)
    xn = (1 - my_x, my_y, my_z)
    yn = (my_x, 1 - my_y, my_z)
    hbase = my_z * HD2
    rbase = my_z * (H * Dr // 2)

    def gq_cols(k):
        return pl.ds(hbase + k * GW, GW)

    def gq_rcols(k):
        return pl.ds(rbase + k * GRW, GRW)

    def grem_cols(k):
        return pl.ds((HD2 - hbase) + k * GW, GW)

    def wq_fetch(k, sl):
        pltpu.make_async_copy(
            wq_hbm.at[:, gq_cols(k)], wq_stage.at[sl], wq_sems.at[sl]
        ).start()
        pltpu.make_async_copy(
            wqr_hbm.at[:, gq_rcols(k)], wqr_stage.at[sl], wqr_sems.at[sl]
        ).start()

    def wq_wait(sl):
        pltpu.make_async_copy(
            wq_hbm.at[:, gq_cols(0)], wq_stage.at[sl], wq_sems.at[sl]
        ).wait()
        pltpu.make_async_copy(
            wqr_hbm.at[:, gq_rcols(0)], wqr_stage.at[sl], wqr_sems.at[sl]
        ).wait()

    def o_rdma(k, mine):
        cols = gq_cols(k) if mine else grem_cols(k)
        return pltpu.make_async_remote_copy(
            src_ref=o_all.at[:, cols], dst_ref=o_all.at[:, cols],
            send_sem=osend.at[k], recv_sem=orecv.at[k],
            device_id=zp, device_id_type=_MESH,
        )

    def z_rdmas():
        pairs = ((c_loc, c_rem), (wuk_loc, wuk_rem), (wuv_loc, wuv_rem))
        return [
            pltpu.make_async_remote_copy(
                src_ref=src, dst_ref=dst,
                send_sem=zsend.at[i], recv_sem=zrecv.at[i],
                device_id=zp, device_id_type=_MESH,
            )
            for i, (src, dst) in enumerate(pairs)
        ]

    def jcols(j):
        return pl.ds(j * SW, SW)

    def stripe_rdmas(j):
        r1 = pltpu.make_async_remote_copy(
            src_ref=out_ref.at[q, :, jcols(j)],
            dst_ref=out_ref.at[q, :, jcols(j)],
            send_sem=gsend.at[0, j], recv_sem=grecv.at[0, j],
            device_id=xn, device_id_type=_MESH,
        )
        r2 = pltpu.make_async_remote_copy(
            src_ref=out_ref.at[q, :, jcols(j)],
            dst_ref=out_ref.at[q, :, jcols(j)],
            send_sem=gsend.at[1, j], recv_sem=grecv.at[1, j],
            device_id=yn, device_id_type=_MESH,
        )
        return r1, r2

    def forward_rdmas(j):
        S2 = S // 2
        fy = pltpu.make_async_remote_copy(
            src_ref=out_ref.at[q_xn, pl.ds(0, S2), jcols(j)],
            dst_ref=out_ref.at[q_xn, pl.ds(0, S2), jcols(j)],
            send_sem=gsend.at[2, j], recv_sem=grecv.at[2, j],
            device_id=yn, device_id_type=_MESH,
        )
        fx = pltpu.make_async_remote_copy(
            src_ref=out_ref.at[q_yn, pl.ds(S2, S2), jcols(j)],
            dst_ref=out_ref.at[q_yn, pl.ds(S2, S2), jcols(j)],
            send_sem=gsend.at[3, j], recv_sem=grecv.at[3, j],
            device_id=xn, device_id_type=_MESH,
        )
        return fy, fx

    wq_fetch(0, 0)
    cp = pltpu.make_async_copy(x_hbm.at[q], xq32, copy_sem)
    cp.start()
    cp.wait()
    xb[...] = xq32[...].astype(BF16)
    c_loc[...] = jnp.dot(
        xb[...], wdkv_ref[...].astype(BF16), preferred_element_type=F32
    ).astype(BF16)
    their_cols = pl.ds((HD2 - hbase), HD2)
    my_cols = pl.ds(hbase, HD2)
    wuk_loc[...] = wuk_ref[:, their_cols].astype(BF16)
    wuv_loc[...] = wuv_ref[:, their_cols].astype(BF16)

    barrier = pltpu.get_barrier_semaphore()
    for nbr in (zp, xn, yn):
        pl.semaphore_signal(barrier, inc=1, device_id=nbr,
                            device_id_type=_MESH)
    pl.semaphore_wait(barrier, 3)

    zr = z_rdmas()
    for r in zr:
        r.start()

    kr_scr[...] = jnp.dot(
        xb[...], wkr_ref[...].astype(BF16), preferred_element_type=F32
    ).astype(BF16)

    for k in range(GPZ):
        sl = k % 2
        wq_wait(sl)
        if k + 1 < GPZ:
            wq_fetch(k + 1, 1 - sl)
        qall[:, pl.ds(k * GW, GW)] = jnp.dot(
            xb[...], wq_stage[sl].astype(BF16), preferred_element_type=F32
        ).astype(BF16)
        qrall[:, pl.ds(k * GRW, GRW)] = jnp.dot(
            xb[...], wqr_stage[sl].astype(BF16), preferred_element_type=F32
        ).astype(BF16)

    for r in zr:
        r.wait()
    k_ = jnp.dot(c_loc[...], wuk_ref[:, my_cols].astype(BF16),
                 preferred_element_type=F32)
    k_ += jnp.dot(c_rem[...], wuk_rem[...], preferred_element_type=F32)
    kbuf[...] = k_.astype(BF16)
    v_ = jnp.dot(c_loc[...], wuv_ref[:, my_cols].astype(BF16),
                 preferred_element_type=F32)
    v_ += jnp.dot(c_rem[...], wuv_rem[...], preferred_element_type=F32)
    vbuf[...] = v_.astype(BF16)

    for sl in range(2):
        pltpu.make_async_copy(
            wo_hbm.at[:, jcols(sl)], wo_stage.at[sl], wo_sems.at[sl]
        ).start()

    kr_t = kr_scr[...].T
    for k in range(GPZ):
        qr_g = qrall[:, pl.ds(k * GRW, GRW)]
        outs = []
        for hh in range(GW // Dh):
            col = pl.ds(k * GW + hh * Dh, Dh)
            q_h = qall[:, col]
            qr_h = qr_g[:, hh * Dr:(hh + 1) * Dr]
            k_h = kbuf[:, col]
            v_h = vbuf[:, col]
            s = jnp.dot(q_h, k_h.T, preferred_element_type=F32)
            s += jnp.dot(qr_h, kr_t, preferred_element_type=F32)
            s *= SCALE
            m = jnp.max(s, axis=-1, keepdims=True)
            p = jnp.exp(s - m)
            p = p / jnp.sum(p, axis=-1, keepdims=True)
            outs.append(
                jnp.dot(p.astype(BF16), v_h, preferred_element_type=F32)
            )
        o_all[:, gq_cols(k)] = jnp.concatenate(outs, axis=-1).astype(BF16)
        o_rdma(k, mine=True).start()

    for k in range(GPZ):
        o_rdma(k, mine=False).wait_recv()

    for j in range(NJ):
        sl = j % 2
        pltpu.make_async_copy(
            wo_hbm.at[:, jcols(j)], wo_stage.at[sl], wo_sems.at[sl]
        ).wait()
        oblk = jnp.dot(
            o_all[...], wo_stage[sl].astype(BF16),
            preferred_element_type=F32,
        )
        if j + 2 < NJ:
            pltpu.make_async_copy(
                wo_hbm.at[:, jcols(j + 2)], wo_stage.at[sl],
                wo_sems.at[sl],
            ).start()
        out_ref[pl.ds(q, 1), :, jcols(j)] = (
            oblk.astype(BF16).reshape(1, S, SW)
        )
        r1, r2 = stripe_rdmas(j)
        r1.start()
        r2.start()
        if j > 1:
            p1, p2 = stripe_rdmas(j - 2)
            p1.wait_recv()
            p2.wait_recv()
            fy, fx = forward_rdmas(j - 2)
            fy.start()
            fx.start()

    for j in (NJ - 2, NJ - 1):
        p1, p2 = stripe_rdmas(j)
        p1.wait_recv()
        p2.wait_recv()
        fy, fx = forward_rdmas(j)
        fy.start()
        fx.start()
    for j in range(NJ):
        fy, fx = forward_rdmas(j)
        fy.wait_recv()
        fx.wait_recv()
    for j in range(NJ):
        r1, r2 = stripe_rdmas(j)
        r1.wait_send()
        r2.wait_send()
        fy, fx = forward_rdmas(j)
        fy.wait_send()
        fx.wait_send()
    for k in range(GPZ):
        o_rdma(k, mine=True).wait_send()


def kernel(x, Wdkv, Wuk, Wuv, Wq, Wqr, Wkr, Wo):
    return pl.pallas_call(
        _main_body,
        out_shape=jax.ShapeDtypeStruct((B, S, D), BF16),
        in_specs=[
            pl.BlockSpec(memory_space=pl.ANY),
            pl.BlockSpec(memory_space=pltpu.VMEM),
            pl.BlockSpec(memory_space=pltpu.VMEM),
            pl.BlockSpec(memory_space=pltpu.VMEM),
            pl.BlockSpec(memory_space=pltpu.VMEM),
            pl.BlockSpec(memory_space=pl.ANY),
            pl.BlockSpec(memory_space=pl.ANY),
            pl.BlockSpec(memory_space=pl.ANY),
        ],
        out_specs=pl.BlockSpec(memory_space=pltpu.VMEM),
        scratch_shapes=[
            pltpu.VMEM((S, D), F32),
            pltpu.VMEM((S, D), BF16),
            pltpu.VMEM((S, DCS), BF16),
            pltpu.VMEM((S, DCS), BF16),
            pltpu.VMEM((DCS, HD2), BF16),
            pltpu.VMEM((DCS, HD2), BF16),
            pltpu.VMEM((DCS, HD2), BF16),
            pltpu.VMEM((DCS, HD2), BF16),
            pltpu.VMEM((S, HD2), BF16),
            pltpu.VMEM((S, HD2), BF16),
            pltpu.VMEM((S, Dr), BF16),
            pltpu.VMEM((S, HD2), BF16),
            pltpu.VMEM((S, H * Dr // 2), BF16),
            pltpu.VMEM((S, HD), BF16),
            pltpu.VMEM((2, D, GW), F32),
            pltpu.VMEM((2, D, GRW), F32),
            pltpu.VMEM((2, D, SW), F32),
            pltpu.SemaphoreType.DMA,
            pltpu.SemaphoreType.DMA((2,)),
            pltpu.SemaphoreType.DMA((2,)),
            pltpu.SemaphoreType.DMA((2,)),
            pltpu.SemaphoreType.DMA((3,)),
            pltpu.SemaphoreType.DMA((3,)),
            pltpu.SemaphoreType.DMA((GPZ,)),
            pltpu.SemaphoreType.DMA((GPZ,)),
            pltpu.SemaphoreType.DMA((4, NJ)),
            pltpu.SemaphoreType.DMA((4, NJ)),
        ],
        compiler_params=pltpu.CompilerParams(
            collective_id=0,
            vmem_limit_bytes=62 * 1024 * 1024,
        ),
    )(x, Wdkv, Wuk, Wuv, Wkr, Wq, Wqr, Wo)


# device time: 102637 ns/iter; 1.0550x vs baseline; 1.0550x over previous
import jax
import jax.numpy as jnp
from jax import lax
from jax.experimental import pallas as pl
from jax.experimental.pallas import tpu as pltpu

B, S, H, Dh, Dr = 4, 256, 32, 128, 64
D = 4096
DCS = 128
HD = H * Dh
HD2 = HD // 2
SCALE = (Dh + Dr) ** -0.5
BF16 = jnp.bfloat16
F32 = jnp.float32
GPZ = 8
GW = HD2 // GPZ
GRW = GW // Dh * Dr
NJ = 16
SW = D // NJ
SW2 = SW // 2

_MESH = pl.DeviceIdType.MESH


def _main_body(x_hbm, wdkv_ref, wuk_ref, wuv_ref, wkr_ref,
               wq_hbm, wqr_hbm, wo_hbm, out_ref,
               xq32, xb, c_loc, c_rem, wuk_loc, wuk_rem, wuv_loc, wuv_rem,
               kbuf, vbuf, kr_scr, qall, qrall, o_all,
               wq_stage, wqr_stage, wo_stage,
               copy_sem, wq_sems, wqr_sems, wo_sems,
               zsend, zrecv, osend, orecv, gsend, grecv):
    my_x = lax.axis_index("x")
    my_y = lax.axis_index("y")
    my_z = lax.axis_index("z")
    q = 2 * my_x + my_y
    q_xn = 2 * (1 - my_x) + my_y
    q_yn = 2 * my_x + (1 - my_y)
    zp = (my_x, my_y, 1 - my_z)
    xn = (1 - my_x, my_y, my_z)
    yn = (my_x, 1 - my_y, my_z)
    hbase = my_z * HD2
    rbase = my_z * (H * Dr // 2)

    def gq_cols(k):
        return pl.ds(hbase + k * GW, GW)

    def gq_rcols(k):
        return pl.ds(rbase + k * GRW, GRW)

    def grem_cols(k):
        return pl.ds((HD2 - hbase) + k * GW, GW)

    def wq_fetch(k, sl):
        pltpu.make_async_copy(
            wq_hbm.at[:, gq_cols(k)], wq_stage.at[sl], wq_sems.at[sl]
        ).start()
        pltpu.make_async_copy(
            wqr_hbm.at[:, gq_rcols(k)], wqr_stage.at[sl], wqr_sems.at[sl]
        ).start()

    def wq_wait(sl):
        pltpu.make_async_copy(
            wq_hbm.at[:, gq_cols(0)], wq_stage.at[sl], wq_sems.at[sl]
        ).wait()
        pltpu.make_async_copy(
            wqr_hbm.at[:, gq_rcols(0)], wqr_stage.at[sl], wqr_sems.at[sl]
        ).wait()

    def o_rdma(k, mine):
        cols = gq_cols(k) if mine else grem_cols(k)
        return pltpu.make_async_remote_copy(
            src_ref=o_all.at[:, cols], dst_ref=o_all.at[:, cols],
            send_sem=osend.at[k], recv_sem=orecv.at[k],
            device_id=zp, device_id_type=_MESH,
        )

    def z_rdmas():
        pairs = ((c_loc, c_rem), (wuk_loc, wuk_rem), (wuv_loc, wuv_rem))
        return [
            pltpu.make_async_remote_copy(
                src_ref=src, dst_ref=dst,
                send_sem=zsend.at[i], recv_sem=zrecv.at[i],
                device_id=zp, device_id_type=_MESH,
            )
            for i, (src, dst) in enumerate(pairs)
        ]

    def jcols(j):
        return pl.ds(j * SW, SW)

    def stripe_rdmas(j):
        r1 = pltpu.make_async_remote_copy(
            src_ref=out_ref.at[q, :, jcols(j)],
            dst_ref=out_ref.at[q, :, jcols(j)],
            send_sem=gsend.at[0, j], recv_sem=grecv.at[0, j],
            device_id=xn, device_id_type=_MESH,
        )
        r2 = pltpu.make_async_remote_copy(
            src_ref=out_ref.at[q, :, jcols(j)],
            dst_ref=out_ref.at[q, :, jcols(j)],
            send_sem=gsend.at[1, j], recv_sem=grecv.at[1, j],
            device_id=yn, device_id_type=_MESH,
        )
        return r1, r2

    def forward_rdmas(j):
        S2 = S // 2
        fy = pltpu.make_async_remote_copy(
            src_ref=out_ref.at[q_xn, pl.ds(0, S2), jcols(j)],
            dst_ref=out_ref.at[q_xn, pl.ds(0, S2), jcols(j)],
            send_sem=gsend.at[2, j], recv_sem=grecv.at[2, j],
            device_id=yn, device_id_type=_MESH,
        )
        fx = pltpu.make_async_remote_copy(
            src_ref=out_ref.at[q_yn, pl.ds(S2, S2), jcols(j)],
            dst_ref=out_ref.at[q_yn, pl.ds(S2, S2), jcols(j)],
            send_sem=gsend.at[3, j], recv_sem=grecv.at[3, j],
            device_id=xn, device_id_type=_MESH,
        )
        return fy, fx

    wq_fetch(0, 0)
    cp = pltpu.make_async_copy(x_hbm.at[q], xq32, copy_sem)
    cp.start()
    cp.wait()
    xb[...] = xq32[...].astype(BF16)
    c_loc[...] = jnp.dot(
        xb[...], wdkv_ref[...].astype(BF16), preferred_element_type=F32
    ).astype(BF16)
    their_cols = pl.ds((HD2 - hbase), HD2)
    my_cols = pl.ds(hbase, HD2)
    wuk_loc[...] = wuk_ref[:, their_cols].astype(BF16)
    wuv_loc[...] = wuv_ref[:, their_cols].astype(BF16)

    barrier = pltpu.get_barrier_semaphore()
    for nbr in (zp, xn, yn):
        pl.semaphore_signal(barrier, inc=1, device_id=nbr,
                            device_id_type=_MESH)
    pl.semaphore_wait(barrier, 3)

    zr = z_rdmas()
    for r in zr:
        r.start()

    kr_scr[...] = jnp.dot(
        xb[...], wkr_ref[...].astype(BF16), preferred_element_type=F32
    ).astype(BF16)

    for k in range(GPZ):
        sl = k % 2
        wq_wait(sl)
        if k + 1 < GPZ:
            wq_fetch(k + 1, 1 - sl)
        qall[:, pl.ds(k * GW, GW)] = jnp.dot(
            xb[...], wq_stage[sl].astype(BF16), preferred_element_type=F32
        ).astype(BF16)
        qrall[:, pl.ds(k * GRW, GRW)] = jnp.dot(
            xb[...], wqr_stage[sl].astype(BF16), preferred_element_type=F32
        ).astype(BF16)

    for r in zr:
        r.wait()
    k_ = jnp.dot(c_loc[...], wuk_ref[:, my_cols].astype(BF16),
                 preferred_element_type=F32)
    k_ += jnp.dot(c_rem[...], wuk_rem[...], preferred_element_type=F32)
    kbuf[...] = k_.astype(BF16)
    v_ = jnp.dot(c_loc[...], wuv_ref[:, my_cols].astype(BF16),
                 preferred_element_type=F32)
    v_ += jnp.dot(c_rem[...], wuv_rem[...], preferred_element_type=F32)
    vbuf[...] = v_.astype(BF16)

    for sl in range(2):
        pltpu.make_async_copy(
            wo_hbm.at[:, jcols(sl)], wo_stage.at[sl], wo_sems.at[sl]
        ).start()

    kr_t = kr_scr[...].T
    for k in range(GPZ):
        qr_g = qrall[:, pl.ds(k * GRW, GRW)]
        outs = []
        for hh in range(GW // Dh):
            col = pl.ds(k * GW + hh * Dh, Dh)
            q_h = qall[:, col]
            qr_h = qr_g[:, hh * Dr:(hh + 1) * Dr]
            k_h = kbuf[:, col]
            v_h = vbuf[:, col]
            s = jnp.dot(q_h, k_h.T, preferred_element_type=F32)
            s += jnp.dot(qr_h, kr_t, preferred_element_type=F32)
            s *= SCALE
            m = jnp.max(s, axis=-1, keepdims=True)
            p = jnp.exp(s - m)
            p = p / jnp.sum(p, axis=-1, keepdims=True)
            outs.append(
                jnp.dot(p.astype(BF16), v_h, preferred_element_type=F32)
            )
        o_all[:, gq_cols(k)] = jnp.concatenate(outs, axis=-1).astype(BF16)
        o_rdma(k, mine=True).start()

    for k in range(GPZ):
        o_rdma(k, mine=False).wait_recv()

    for j in range(NJ):
        sl = j % 2
        pltpu.make_async_copy(
            wo_hbm.at[:, jcols(j)], wo_stage.at[sl], wo_sems.at[sl]
        ).wait()
        oblk = jnp.dot(
            o_all[...], wo_stage[sl].astype(BF16),
            preferred_element_type=F32,
        )
        if j + 2 < NJ:
            pltpu.make_async_copy(
                wo_hbm.at[:, jcols(j + 2)], wo_stage.at[sl],
                wo_sems.at[sl],
            ).start()
        out_ref[pl.ds(q, 1), :, jcols(j)] = (
            oblk.astype(BF16).reshape(1, S, SW)
        )
        r1, r2 = stripe_rdmas(j)
        r1.start()
        r2.start()
        if j > 1:
            p1, p2 = stripe_rdmas(j - 2)
            p1.wait_recv()
            p2.wait_recv()
            fy, fx = forward_rdmas(j - 2)
            fy.start()
            fx.start()

    for j in (NJ - 2, NJ - 1):
        p1, p2 = stripe_rdmas(j)
        p1.wait_recv()
        p2.wait_recv()
        fy, fx = forward_rdmas(j)
        fy.start()
        fx.start()
    for j in range(NJ):
        fy, fx = forward_rdmas(j)
        fy.wait_recv()
        fx.wait_recv()
    for j in range(NJ):
        r1, r2 = stripe_rdmas(j)
        r1.wait_send()
        r2.wait_send()
        fy, fx = forward_rdmas(j)
        fy.wait_send()
        fx.wait_send()
    for k in range(GPZ):
        o_rdma(k, mine=True).wait_send()


def kernel(x, Wdkv, Wuk, Wuv, Wq, Wqr, Wkr, Wo):
    return pl.pallas_call(
        _main_body,
        out_shape=jax.ShapeDtypeStruct((B, S, D), BF16),
        in_specs=[
            pl.BlockSpec(memory_space=pl.ANY),
            pl.BlockSpec(memory_space=pltpu.VMEM),
            pl.BlockSpec(memory_space=pltpu.VMEM),
            pl.BlockSpec(memory_space=pltpu.VMEM),
            pl.BlockSpec(memory_space=pltpu.VMEM),
            pl.BlockSpec(memory_space=pl.ANY),
            pl.BlockSpec(memory_space=pl.ANY),
            pl.BlockSpec(memory_space=pl.ANY),
        ],
        out_specs=pl.BlockSpec(memory_space=pltpu.VMEM),
        scratch_shapes=[
            pltpu.VMEM((S, D), F32),
            pltpu.VMEM((S, D), BF16),
            pltpu.VMEM((S, DCS), BF16),
            pltpu.VMEM((S, DCS), BF16),
            pltpu.VMEM((DCS, HD2), BF16),
            pltpu.VMEM((DCS, HD2), BF16),
            pltpu.VMEM((DCS, HD2), BF16),
            pltpu.VMEM((DCS, HD2), BF16),
            pltpu.VMEM((S, HD2), BF16),
            pltpu.VMEM((S, HD2), BF16),
            pltpu.VMEM((S, Dr), BF16),
            pltpu.VMEM((S, HD2), BF16),
            pltpu.VMEM((S, H * Dr // 2), BF16),
            pltpu.VMEM((S, HD), BF16),
            pltpu.VMEM((2, D, GW), F32),
            pltpu.VMEM((2, D, GRW), F32),
            pltpu.VMEM((2, D, SW), F32),
            pltpu.SemaphoreType.DMA,
            pltpu.SemaphoreType.DMA((2,)),
            pltpu.SemaphoreType.DMA((2,)),
            pltpu.SemaphoreType.DMA((2,)),
            pltpu.SemaphoreType.DMA((3,)),
            pltpu.SemaphoreType.DMA((3,)),
            pltpu.SemaphoreType.DMA((GPZ,)),
            pltpu.SemaphoreType.DMA((GPZ,)),
            pltpu.SemaphoreType.DMA((4, NJ)),
            pltpu.SemaphoreType.DMA((4, NJ)),
        ],
        compiler_params=pltpu.CompilerParams(
            collective_id=0,
            vmem_limit_bytes=62 * 1024 * 1024,
        ),
    )(x, Wdkv, Wuk, Wuv, Wkr, Wq, Wqr, Wo)


# device time: 99444 ns/iter; 1.0888x vs baseline; 1.0321x over previous
import jax
import jax.numpy as jnp
from jax import lax
from jax.experimental import pallas as pl
from jax.experimental.pallas import tpu as pltpu

B, S, H, Dh, Dr = 4, 256, 32, 128, 64
D = 4096
DCS = 128
HD = H * Dh
HD2 = HD // 2
SCALE = (Dh + Dr) ** -0.5
BF16 = jnp.bfloat16
F32 = jnp.float32
GPZ = 8
GW = HD2 // GPZ
GRW = GW // Dh * Dr
NJ = 16
SW = D // NJ
SW2 = SW // 2

_MESH = pl.DeviceIdType.MESH


def _main_body(x_hbm, wdkv_ref, wuk_ref, wuv_ref, wkr_ref,
               wq_hbm, wqr_hbm, wo_hbm, out_ref,
               xq32, xb, c_loc, c_rem, wuk_loc, wuk_rem, wuv_loc, wuv_rem,
               kbuf, vbuf, kr_scr, qall, qrall, o_all,
               wq_stage, wqr_stage, wo_stage,
               copy_sem, wq_sems, wqr_sems, wo_sems,
               zsend, zrecv, osend, orecv, gsend, grecv):
    my_x = lax.axis_index("x")
    my_y = lax.axis_index("y")
    my_z = lax.axis_index("z")
    q = 2 * my_x + my_y
    q_xn = 2 * (1 - my_x) + my_y
    q_yn = 2 * my_x + (1 - my_y)
    zp = (my_x, my_y, 1 - my_z)
    xn = (1 - my_x, my_y, my_z)
    yn = (my_x, 1 - my_y, my_z)
    hbase = my_z * HD2
    rbase = my_z * (H * Dr // 2)

    def gq_cols(k):
        return pl.ds(hbase + k * GW, GW)

    def gq_rcols(k):
        return pl.ds(rbase + k * GRW, GRW)

    def grem_cols(k):
        return pl.ds((HD2 - hbase) + k * GW, GW)

    def wq_fetch(k, sl):
        pltpu.make_async_copy(
            wq_hbm.at[:, gq_cols(k)], wq_stage.at[sl], wq_sems.at[sl]
        ).start()
        pltpu.make_async_copy(
            wqr_hbm.at[:, gq_rcols(k)], wqr_stage.at[sl], wqr_sems.at[sl]
        ).start()

    def wq_wait(sl):
        pltpu.make_async_copy(
            wq_hbm.at[:, gq_cols(0)], wq_stage.at[sl], wq_sems.at[sl]
        ).wait()
        pltpu.make_async_copy(
            wqr_hbm.at[:, gq_rcols(0)], wqr_stage.at[sl], wqr_sems.at[sl]
        ).wait()

    def o_rdma(k, mine):
        cols = gq_cols(k) if mine else grem_cols(k)
        return pltpu.make_async_remote_copy(
            src_ref=o_all.at[:, cols], dst_ref=o_all.at[:, cols],
            send_sem=osend.at[k], recv_sem=orecv.at[k],
            device_id=zp, device_id_type=_MESH,
        )

    def z_rdmas():
        pairs = ((c_loc, c_rem), (wuk_loc, wuk_rem), (wuv_loc, wuv_rem))
        return [
            pltpu.make_async_remote_copy(
                src_ref=src, dst_ref=dst,
                send_sem=zsend.at[i], recv_sem=zrecv.at[i],
                device_id=zp, device_id_type=_MESH,
            )
            for i, (src, dst) in enumerate(pairs)
        ]

    def jcols(j):
        return pl.ds(j * SW, SW)

    def stripe_rdmas(j):
        r1 = pltpu.make_async_remote_copy(
            src_ref=out_ref.at[q, :, jcols(j)],
            dst_ref=out_ref.at[q, :, jcols(j)],
            send_sem=gsend.at[0, j], recv_sem=grecv.at[0, j],
            device_id=xn, device_id_type=_MESH,
        )
        r2 = pltpu.make_async_remote_copy(
            src_ref=out_ref.at[q, :, jcols(j)],
            dst_ref=out_ref.at[q, :, jcols(j)],
            send_sem=gsend.at[1, j], recv_sem=grecv.at[1, j],
            device_id=yn, device_id_type=_MESH,
        )
        return r1, r2

    def forward_rdmas(j):
        S2 = S // 2
        fy = pltpu.make_async_remote_copy(
            src_ref=out_ref.at[q_xn, pl.ds(0, S2), jcols(j)],
            dst_ref=out_ref.at[q_xn, pl.ds(0, S2), jcols(j)],
            send_sem=gsend.at[2, j], recv_sem=grecv.at[2, j],
            device_id=yn, device_id_type=_MESH,
        )
        fx = pltpu.make_async_remote_copy(
            src_ref=out_ref.at[q_yn, pl.ds(S2, S2), jcols(j)],
            dst_ref=out_ref.at[q_yn, pl.ds(S2, S2), jcols(j)],
            send_sem=gsend.at[3, j], recv_sem=grecv.at[3, j],
            device_id=xn, device_id_type=_MESH,
        )
        return fy, fx

    wq_fetch(0, 0)
    cp = pltpu.make_async_copy(x_hbm.at[q], xq32, copy_sem)
    cp.start()
    cp.wait()
    xb[...] = xq32[...].astype(BF16)
    c_loc[...] = jnp.dot(
        xb[...], wdkv_ref[...].astype(BF16), preferred_element_type=F32
    ).astype(BF16)
    their_cols = pl.ds((HD2 - hbase), HD2)
    my_cols = pl.ds(hbase, HD2)
    wuk_loc[...] = wuk_ref[:, their_cols].astype(BF16)
    wuv_loc[...] = wuv_ref[:, their_cols].astype(BF16)

    barrier = pltpu.get_barrier_semaphore()
    for nbr in (zp, xn, yn):
        pl.semaphore_signal(barrier, inc=1, device_id=nbr,
                            device_id_type=_MESH)
    pl.semaphore_wait(barrier, 3)

    zr = z_rdmas()
    for r in zr:
        r.start()

    kr_scr[...] = jnp.dot(
        xb[...], wkr_ref[...].astype(BF16).T, preferred_element_type=F32
    ).astype(BF16)

    for k in range(GPZ):
        sl = k % 2
        wq_wait(sl)
        if k + 1 < GPZ:
            wq_fetch(k + 1, 1 - sl)
        qall[:, pl.ds(k * GW, GW)] = jnp.dot(
            xb[...], wq_stage[sl].astype(BF16), preferred_element_type=F32
        ).astype(BF16)
        qrall[:, pl.ds(k * GRW, GRW)] = jnp.dot(
            xb[...], wqr_stage[sl].astype(BF16), preferred_element_type=F32
        ).astype(BF16)

    for r in zr:
        r.wait()
    k_ = jnp.dot(c_loc[...], wuk_ref[:, my_cols].astype(BF16),
                 preferred_element_type=F32)
    k_ += jnp.dot(c_rem[...], wuk_rem[...], preferred_element_type=F32)
    kbuf[...] = k_.astype(BF16)
    v_ = jnp.dot(c_loc[...], wuv_ref[:, my_cols].astype(BF16),
                 preferred_element_type=F32)
    v_ += jnp.dot(c_rem[...], wuv_rem[...], preferred_element_type=F32)
    vbuf[...] = v_.astype(BF16)

    for sl in range(2):
        pltpu.make_async_copy(
            wo_hbm.at[:, jcols(sl)], wo_stage.at[sl], wo_sems.at[sl]
        ).start()

    kr_t = kr_scr[...].T
    for k in range(GPZ):
        qr_g = qrall[:, pl.ds(k * GRW, GRW)]
        outs = []
        for hh in range(GW // Dh):
            col = pl.ds(k * GW + hh * Dh, Dh)
            q_h = qall[:, col]
            qr_h = qr_g[:, hh * Dr:(hh + 1) * Dr]
            k_h = kbuf[:, col]
            v_h = vbuf[:, col]
            s = jnp.dot(q_h, k_h.T, preferred_element_type=F32)
            s += jnp.dot(qr_h, kr_t, preferred_element_type=F32)
            s *= SCALE
            m = jnp.max(s, axis=-1, keepdims=True)
            p = jnp.exp(s - m)
            p = p / jnp.sum(p, axis=-1, keepdims=True)
            outs.append(
                jnp.dot(p.astype(BF16), v_h, preferred_element_type=F32)
            )
        o_all[:, gq_cols(k)] = jnp.concatenate(outs, axis=-1).astype(BF16)
        o_rdma(k, mine=True).start()

    for k in range(GPZ):
        o_rdma(k, mine=False).wait_recv()

    for j in range(NJ):
        sl = j % 2
        pltpu.make_async_copy(
            wo_hbm.at[:, jcols(j)], wo_stage.at[sl], wo_sems.at[sl]
        ).wait()
        oblk = jnp.dot(
            o_all[...], wo_stage[sl].astype(BF16),
            preferred_element_type=F32,
        )
        if j + 2 < NJ:
            pltpu.make_async_copy(
                wo_hbm.at[:, jcols(j + 2)], wo_stage.at[sl],
                wo_sems.at[sl],
            ).start()
        out_ref[pl.ds(q, 1), :, jcols(j)] = (
            oblk.astype(BF16).reshape(1, S, SW)
        )
        r1, r2 = stripe_rdmas(j)
        r1.start()
        r2.start()
        if j > 1:
            p1, p2 = stripe_rdmas(j - 2)
            p1.wait_recv()
            p2.wait_recv()
            fy, fx = forward_rdmas(j - 2)
            fy.start()
            fx.start()

    for j in (NJ - 2, NJ - 1):
        p1, p2 = stripe_rdmas(j)
        p1.wait_recv()
        p2.wait_recv()
        fy, fx = forward_rdmas(j)
        fy.start()
        fx.start()
    for j in range(NJ):
        fy, fx = forward_rdmas(j)
        fy.wait_recv()
        fx.wait_recv()
    for j in range(NJ):
        r1, r2 = stripe_rdmas(j)
        r1.wait_send()
        r2.wait_send()
        fy, fx = forward_rdmas(j)
        fy.wait_send()
        fx.wait_send()
    for k in range(GPZ):
        o_rdma(k, mine=True).wait_send()


def kernel(x, Wdkv, Wuk, Wuv, Wq, Wqr, Wkr, Wo):
    return pl.pallas_call(
        _main_body,
        out_shape=jax.ShapeDtypeStruct((B, S, D), BF16),
        in_specs=[
            pl.BlockSpec(memory_space=pl.ANY),
            pl.BlockSpec(memory_space=pltpu.VMEM),
            pl.BlockSpec(memory_space=pltpu.VMEM),
            pl.BlockSpec(memory_space=pltpu.VMEM),
            pl.BlockSpec(memory_space=pltpu.VMEM),
            pl.BlockSpec(memory_space=pl.ANY),
            pl.BlockSpec(memory_space=pl.ANY),
            pl.BlockSpec(memory_space=pl.ANY),
        ],
        out_specs=pl.BlockSpec(memory_space=pltpu.VMEM),
        scratch_shapes=[
            pltpu.VMEM((S, D), F32),
            pltpu.VMEM((S, D), BF16),
            pltpu.VMEM((S, DCS), BF16),
            pltpu.VMEM((S, DCS), BF16),
            pltpu.VMEM((DCS, HD2), BF16),
            pltpu.VMEM((DCS, HD2), BF16),
            pltpu.VMEM((DCS, HD2), BF16),
            pltpu.VMEM((DCS, HD2), BF16),
            pltpu.VMEM((S, HD2), BF16),
            pltpu.VMEM((S, HD2), BF16),
            pltpu.VMEM((S, Dr), BF16),
            pltpu.VMEM((S, HD2), BF16),
            pltpu.VMEM((S, H * Dr // 2), BF16),
            pltpu.VMEM((S, HD), BF16),
            pltpu.VMEM((2, D, GW), F32),
            pltpu.VMEM((2, D, GRW), F32),
            pltpu.VMEM((2, D, SW), F32),
            pltpu.SemaphoreType.DMA,
            pltpu.SemaphoreType.DMA((2,)),
            pltpu.SemaphoreType.DMA((2,)),
            pltpu.SemaphoreType.DMA((2,)),
            pltpu.SemaphoreType.DMA((3,)),
            pltpu.SemaphoreType.DMA((3,)),
            pltpu.SemaphoreType.DMA((GPZ,)),
            pltpu.SemaphoreType.DMA((GPZ,)),
            pltpu.SemaphoreType.DMA((4, NJ)),
            pltpu.SemaphoreType.DMA((4, NJ)),
        ],
        compiler_params=pltpu.CompilerParams(
            collective_id=0,
            vmem_limit_bytes=62 * 1024 * 1024,
        ),
    )(x, Wdkv, Wuk, Wuv, Wkr.T, Wq, Wqr, Wo)
